# baseline (speedup 1.0000x reference)
import functools

import jax
import jax.numpy as jnp
from jax import lax
from jax.experimental import pallas as pl
from jax.experimental.pallas import tpu as pltpu
from jax.experimental.pallas import tpu_sc as plsc

B = 16384
D = 64
NW = 32
BPW = B // NW
SUB = 128
NSUB = BPW // SUB
BROWS = 7816

_mesh = plsc.VectorSubcoreMesh(core_axis_name="c", subcore_axis_name="s")


@functools.partial(
    pl.kernel,
    mesh=_mesh,
    out_type=jax.ShapeDtypeStruct((B,), jnp.float32),
    scratch_types=[
        pltpu.VMEM((NSUB, SUB), jnp.int32),
        pltpu.VMEM((NSUB, SUB), jnp.int32),
        pltpu.VMEM((NSUB, SUB), jnp.int32),
        pltpu.VMEM((NSUB, SUB), jnp.int32),
        pltpu.VMEM((NSUB, SUB), jnp.int32),
        pltpu.VMEM((NSUB, SUB), jnp.int32),
        pltpu.VMEM((NSUB, SUB), jnp.int32),
        pltpu.VMEM((SUB, 2 * D), jnp.float32),
        pltpu.VMEM((SUB, 2 * D), jnp.float32),
        pltpu.VMEM((SUB, 2 * D), jnp.float32),
        pltpu.VMEM((SUB, 2 * D), jnp.float32),
        pltpu.VMEM((1, D), jnp.float32),
        pltpu.VMEM((BPW,), jnp.float32),
        pltpu.SemaphoreType.DMA,
    ],
    compiler_params=pltpu.CompilerParams(
        needs_layout_passes=False, use_tc_tiling_on_sc=False),
)
def _trans_rec_sc(uids_hbm, lids_hbm, pids_hbm, ut_hbm, it_hbm, g_hbm, bias_hbm,
                  out_hbm, uid_v, lid_v, pid_v, ur_v, lr_v, pr_v, br_v,
                  u_v, l_v, p_v, b_v, g_v, o_v, sem):
    wid = lax.axis_index("s") * 2 + lax.axis_index("c")
    base = wid * BPW

    pltpu.sync_copy(uids_hbm.at[pl.ds(wid * NSUB, NSUB), :], uid_v)
    pltpu.sync_copy(lids_hbm.at[pl.ds(wid * NSUB, NSUB), :], lid_v)
    pltpu.sync_copy(pids_hbm.at[pl.ds(wid * NSUB, NSUB), :], pid_v)
    pltpu.sync_copy(g_hbm, g_v)

    iota16 = lax.iota(jnp.int32, 16)
    zeros16 = jnp.zeros((16,), jnp.int32)

    for j in range(NSUB):
        rfull = jnp.full((16,), j, jnp.int32)
        for k in range(SUB // 16):
            col = iota16 + (k * 16)
            u_id = plsc.load_gather(uid_v, [rfull, col])
            l_id = plsc.load_gather(lid_v, [rfull, col])
            p_id = plsc.load_gather(pid_v, [rfull, col])
            plsc.store_scatter(ur_v, [rfull, col], u_id >> 1)
            plsc.store_scatter(lr_v, [rfull, col], l_id >> 1)
            plsc.store_scatter(pr_v, [rfull, col], p_id >> 1)
            plsc.store_scatter(br_v, [rfull, col], p_id >> 7)

    for j in range(NSUB):
        c1 = pltpu.async_copy(ut_hbm.at[ur_v.at[j]], u_v, sem)
        c2 = pltpu.async_copy(it_hbm.at[lr_v.at[j]], l_v, sem)
        c3 = pltpu.async_copy(it_hbm.at[pr_v.at[j]], p_v, sem)
        c4 = pltpu.async_copy(bias_hbm.at[br_v.at[j]], b_v, sem)
        c1.wait()
        c2.wait()
        c3.wait()
        c4.wait()

        rfull = jnp.full((16,), j, jnp.int32)
        for k in range(SUB // 16):
            rows = iota16 + (k * 16)
            col = rows
            u_id = plsc.load_gather(uid_v, [rfull, col])
            l_id = plsc.load_gather(lid_v, [rfull, col])
            p_id = plsc.load_gather(pid_v, [rfull, col])
            ub = (u_id & 1) << 6
            lb = (l_id & 1) << 6
            pb = (p_id & 1) << 6

            def dim_body(d, acc):
                dcols = jnp.full((16,), d, jnp.int32)
                u = plsc.load_gather(u_v, [rows, ub + dcols])
                l = plsc.load_gather(l_v, [rows, lb + dcols])
                p = plsc.load_gather(p_v, [rows, pb + dcols])
                gd = plsc.load_gather(g_v, [zeros16, dcols])
                diff = (u + l) - (p - gd)
                return acc + diff * diff

            acc = lax.fori_loop(0, D, dim_body, jnp.zeros((16,), jnp.float32))

            bits = plsc.bitcast(acc, jnp.int32)
            y = plsc.bitcast(jnp.int32(0x1FBD1DF5) + (bits >> 1), jnp.float32)
            for _ in range(3):
                y = 0.5 * (y + acc / y)

            bias = plsc.load_gather(b_v, [rows, p_id & 127])
            o_v[pl.ds(j * SUB + k * 16, 16)] = bias - y

    pltpu.sync_copy(o_v, out_hbm.at[pl.ds(base, BPW)])


def kernel(user_ids, last_items, pre_items, user_table, item_table,
           global_transition, item_biases):
    uid = user_ids.astype(jnp.int32).reshape(B // SUB, SUB)
    lid = last_items.astype(jnp.int32).reshape(B // SUB, SUB)
    pid = pre_items.astype(jnp.int32).reshape(B // SUB, SUB)
    ut2 = user_table.reshape(-1, 2 * D)
    it2 = item_table.reshape(-1, 2 * D)
    bias_flat = item_biases.reshape(-1)
    bias2 = jnp.pad(bias_flat, (0, BROWS * SUB - bias_flat.shape[0]))
    bias2 = bias2.reshape(BROWS, SUB)
    return _trans_rec_sc(uid, lid, pid, ut2, it2, global_transition, bias2)


# speedup vs baseline: 2.2289x; 2.2289x over previous
import functools

import jax
import jax.numpy as jnp
from jax import lax
from jax.experimental import pallas as pl
from jax.experimental.pallas import tpu as pltpu
from jax.experimental.pallas import tpu_sc as plsc

B = 16384
D = 64
NW = 32
BPW = B // NW
PC = 16
NPC = BPW // PC
IDR = 4
NBUF = 2
BROWS = 7816

_mesh = plsc.VectorSubcoreMesh(core_axis_name="c", subcore_axis_name="s")


@functools.partial(
    pl.kernel,
    mesh=_mesh,
    out_type=jax.ShapeDtypeStruct((B,), jnp.float32),
    scratch_types=[
        pltpu.VMEM((IDR, 128), jnp.int32),
        pltpu.VMEM((IDR, 128), jnp.int32),
        pltpu.VMEM((IDR, 128), jnp.int32),
        pltpu.VMEM((IDR, 128), jnp.int32),
        pltpu.VMEM((NBUF, PC, 8, D), jnp.float32),
        pltpu.VMEM((NBUF, PC, 8, D), jnp.float32),
        pltpu.VMEM((NBUF, PC, 8, D), jnp.float32),
        pltpu.VMEM((NBUF, PC, 128), jnp.float32),
        pltpu.VMEM((1, D), jnp.float32),
        pltpu.VMEM((BPW,), jnp.float32),
        pltpu.SemaphoreType.DMA,
        pltpu.SemaphoreType.DMA,
    ],
    compiler_params=pltpu.CompilerParams(
        needs_layout_passes=False, use_tc_tiling_on_sc=True),
)
def _trans_rec_sc(uids_hbm, lids_hbm, pids_hbm, ut_hbm, it_hbm, g_hbm, bias_hbm,
                  out_hbm, uid_v, lid_v, pid_v, br_v,
                  u_v, l_v, p_v, b_v, g_v, o_v, sem0, sem1):
    wid = lax.axis_index("s") * 2 + lax.axis_index("c")
    base = wid * BPW

    pltpu.sync_copy(uids_hbm.at[pl.ds(wid * IDR, IDR), :], uid_v)
    pltpu.sync_copy(lids_hbm.at[pl.ds(wid * IDR, IDR), :], lid_v)
    pltpu.sync_copy(pids_hbm.at[pl.ds(wid * IDR, IDR), :], pid_v)
    pltpu.sync_copy(g_hbm, g_v)

    iota16 = lax.iota(jnp.int32, 16)
    zeros16 = jnp.zeros((16,), jnp.int32)
    sems = [sem0, sem1]

    for j in range(IDR):
        rfull = jnp.full((16,), j, jnp.int32)
        for k in range(8):
            col = iota16 + k * 16
            p_id = plsc.load_gather(pid_v, [rfull, col])
            plsc.store_scatter(br_v, [rfull, col], p_id >> 7)

    def fire(p, buf):
        sem = sems[buf]
        r, c = p >> 3, (p & 7) * 16
        sl = pl.ds(c, 16)
        pltpu.async_copy(bias_hbm.at[br_v.at[r, sl]], b_v.at[buf], sem)
        rowp = jnp.full((16,), r, jnp.int32)
        colp = iota16 + c
        u_rows = plsc.load_gather(uid_v, [rowp, colp]) >> 3
        l_rows = plsc.load_gather(lid_v, [rowp, colp]) >> 3
        p_rows = plsc.load_gather(pid_v, [rowp, colp]) >> 3
        for e in range(PC):
            pltpu.async_copy(ut_hbm.at[pl.ds(u_rows[e], 1)],
                             u_v.at[buf, pl.ds(e, 1)], sem)
            pltpu.async_copy(it_hbm.at[pl.ds(l_rows[e], 1)],
                             l_v.at[buf, pl.ds(e, 1)], sem)
            pltpu.async_copy(it_hbm.at[pl.ds(p_rows[e], 1)],
                             p_v.at[buf, pl.ds(e, 1)], sem)

    def drain(buf):
        sem = sems[buf]
        pltpu.make_async_copy(bias_hbm.at[br_v.at[0, pl.ds(0, 16)]],
                              b_v.at[buf], sem).wait()
        for e in range(PC):
            pltpu.make_async_copy(ut_hbm.at[pl.ds(0, 1)],
                                  u_v.at[buf, pl.ds(e, 1)], sem).wait()
            pltpu.make_async_copy(it_hbm.at[pl.ds(0, 1)],
                                  l_v.at[buf, pl.ds(e, 1)], sem).wait()
            pltpu.make_async_copy(it_hbm.at[pl.ds(0, 1)],
                                  p_v.at[buf, pl.ds(e, 1)], sem).wait()

    def compute(p, buf):
        rowp = jnp.full((16,), p >> 3, jnp.int32)
        colp = iota16 + (p & 7) * 16
        u_id = plsc.load_gather(uid_v, [rowp, colp])
        l_id = plsc.load_gather(lid_v, [rowp, colp])
        p_id = plsc.load_gather(pid_v, [rowp, colp])
        us, ls, ps = u_id & 7, l_id & 7, p_id & 7

        def dim_body(d, acc):
            dfull = jnp.full((16,), d, jnp.int32)
            u = plsc.load_gather(u_v.at[buf], [iota16, us, dfull])
            l = plsc.load_gather(l_v.at[buf], [iota16, ls, dfull])
            pe = plsc.load_gather(p_v.at[buf], [iota16, ps, dfull])
            gd = plsc.load_gather(g_v, [zeros16, dfull])
            diff = (u + l) - (pe - gd)
            return acc + diff * diff

        acc = lax.fori_loop(0, D, dim_body, jnp.zeros((16,), jnp.float32))

        bits = plsc.bitcast(acc, jnp.int32)
        y = plsc.bitcast(jnp.int32(0x1FBD1DF5) + (bits >> 1), jnp.float32)
        for _ in range(3):
            y = 0.5 * (y + acc / y)

        bias = plsc.load_gather(b_v.at[buf], [iota16, p_id & 127])
        o_v[pl.ds(p * PC, PC)] = bias - y

    fire(0, 0)
    fire(1, 1)

    def piece_body(i, carry):
        p0 = i * NBUF
        for b in range(NBUF):
            p = p0 + b
            drain(b)
            compute(p, b)

            @pl.when(p + NBUF < NPC)
            def _():
                fire(p + NBUF, b)
        return carry

    lax.fori_loop(0, NPC // NBUF, piece_body, jnp.int32(0))

    pltpu.sync_copy(o_v, out_hbm.at[pl.ds(base, BPW)])


def kernel(user_ids, last_items, pre_items, user_table, item_table,
           global_transition, item_biases):
    uid = user_ids.astype(jnp.int32).reshape(B // 128, 128)
    lid = last_items.astype(jnp.int32).reshape(B // 128, 128)
    pid = pre_items.astype(jnp.int32).reshape(B // 128, 128)
    ut3 = user_table.reshape(125000, 8, D)
    it3 = item_table.reshape(125000, 8, D)
    bias_flat = item_biases.reshape(-1)
    bias2 = jnp.pad(bias_flat, (0, BROWS * 128 - bias_flat.shape[0]))
    bias2 = bias2.reshape(BROWS, 128)
    return _trans_rec_sc(uid, lid, pid, ut3, it3, global_transition, bias2)


# speedup vs baseline: 2.2344x; 1.0025x over previous
import functools

import jax
import jax.numpy as jnp
from jax import lax
from jax.experimental import pallas as pl
from jax.experimental.pallas import tpu as pltpu
from jax.experimental.pallas import tpu_sc as plsc

B = 16384
D = 64
NW = 32
BPW = B // NW
PC = 16
NPC = BPW // PC
IDR = 4
NBUF = 2
BROWS = 7816

_mesh = plsc.VectorSubcoreMesh(core_axis_name="c", subcore_axis_name="s")


@functools.partial(
    pl.kernel,
    mesh=_mesh,
    out_type=jax.ShapeDtypeStruct((B,), jnp.float32),
    scratch_types=[
        pltpu.VMEM((IDR, 128), jnp.int32),
        pltpu.VMEM((IDR, 128), jnp.int32),
        pltpu.VMEM((IDR, 128), jnp.int32),
        pltpu.VMEM((IDR, 128), jnp.int32),
        pltpu.VMEM((NBUF, PC, 8, D), jnp.float32),
        pltpu.VMEM((NBUF, PC, 8, D), jnp.float32),
        pltpu.VMEM((NBUF, PC, 8, D), jnp.float32),
        pltpu.VMEM((NBUF, PC, 128), jnp.float32),
        pltpu.VMEM((1, D), jnp.float32),
        pltpu.VMEM((BPW,), jnp.float32),
        pltpu.SemaphoreType.DMA,
        pltpu.SemaphoreType.DMA,
    ],
    compiler_params=pltpu.CompilerParams(
        needs_layout_passes=False, use_tc_tiling_on_sc=True),
)
def _trans_rec_sc(uids_hbm, lids_hbm, pids_hbm, ut_hbm, it_hbm, g_hbm, bias_hbm,
                  out_hbm, uid_v, lid_v, pid_v, br_v,
                  u_v, l_v, p_v, b_v, g_v, o_v, sem0, sem1):
    wid = lax.axis_index("s") * 2 + lax.axis_index("c")
    base = wid * BPW

    pltpu.sync_copy(uids_hbm.at[pl.ds(wid * IDR, IDR), :], uid_v)
    pltpu.sync_copy(lids_hbm.at[pl.ds(wid * IDR, IDR), :], lid_v)
    pltpu.sync_copy(pids_hbm.at[pl.ds(wid * IDR, IDR), :], pid_v)
    pltpu.sync_copy(g_hbm, g_v)

    iota16 = lax.iota(jnp.int32, 16)
    zeros16 = jnp.zeros((16,), jnp.int32)
    sems = [sem0, sem1]

    for j in range(IDR):
        rfull = jnp.full((16,), j, jnp.int32)
        for k in range(8):
            col = iota16 + k * 16
            p_id = plsc.load_gather(pid_v, [rfull, col])
            plsc.store_scatter(br_v, [rfull, col], p_id >> 7)

    def fire(p, buf):
        sem = sems[buf]
        r, c = p >> 3, (p & 7) * 16
        sl = pl.ds(c, 16)
        pltpu.async_copy(bias_hbm.at[br_v.at[r, sl]], b_v.at[buf], sem)
        rowp = jnp.full((16,), r, jnp.int32)
        colp = iota16 + c
        u_rows = plsc.load_gather(uid_v, [rowp, colp]) >> 3
        l_rows = plsc.load_gather(lid_v, [rowp, colp]) >> 3
        p_rows = plsc.load_gather(pid_v, [rowp, colp]) >> 3
        for e in range(PC):
            pltpu.async_copy(ut_hbm.at[pl.ds(u_rows[e], 1)],
                             u_v.at[buf, pl.ds(e, 1)], sem)
            pltpu.async_copy(it_hbm.at[pl.ds(l_rows[e], 1)],
                             l_v.at[buf, pl.ds(e, 1)], sem)
            pltpu.async_copy(it_hbm.at[pl.ds(p_rows[e], 1)],
                             p_v.at[buf, pl.ds(e, 1)], sem)

    def drain(buf):
        sem = sems[buf]
        pltpu.make_async_copy(bias_hbm.at[br_v.at[0, pl.ds(0, 16)]],
                              b_v.at[buf], sem).wait()
        pltpu.make_async_copy(ut_hbm.at[pl.ds(0, PC)], u_v.at[buf], sem).wait()
        pltpu.make_async_copy(it_hbm.at[pl.ds(0, PC)], l_v.at[buf], sem).wait()
        pltpu.make_async_copy(it_hbm.at[pl.ds(0, PC)], p_v.at[buf], sem).wait()

    def compute(p, buf):
        rowp = jnp.full((16,), p >> 3, jnp.int32)
        colp = iota16 + (p & 7) * 16
        u_id = plsc.load_gather(uid_v, [rowp, colp])
        l_id = plsc.load_gather(lid_v, [rowp, colp])
        p_id = plsc.load_gather(pid_v, [rowp, colp])
        us, ls, ps = u_id & 7, l_id & 7, p_id & 7

        def dim_body(d4, acc):
            for q in range(4):
                dfull = jnp.full((16,), d4 * 4 + q, jnp.int32)
                u = plsc.load_gather(u_v.at[buf], [iota16, us, dfull])
                l = plsc.load_gather(l_v.at[buf], [iota16, ls, dfull])
                pe = plsc.load_gather(p_v.at[buf], [iota16, ps, dfull])
                gd = plsc.load_gather(g_v, [zeros16, dfull])
                diff = (u + l) - (pe - gd)
                acc = acc + diff * diff
            return acc

        acc = lax.fori_loop(0, D // 4, dim_body, jnp.zeros((16,), jnp.float32))

        bits = plsc.bitcast(acc, jnp.int32)
        y = plsc.bitcast(jnp.int32(0x1FBD1DF5) + (bits >> 1), jnp.float32)
        for _ in range(3):
            y = 0.5 * (y + acc / y)

        bias = plsc.load_gather(b_v.at[buf], [iota16, p_id & 127])
        o_v[pl.ds(p * PC, PC)] = bias - y

    fire(0, 0)
    fire(1, 1)

    def piece_body(i, carry):
        p0 = i * NBUF
        for b in range(NBUF):
            p = p0 + b
            drain(b)
            compute(p, b)

            @pl.when(p + NBUF < NPC)
            def _():
                fire(p + NBUF, b)
        return carry

    lax.fori_loop(0, NPC // NBUF, piece_body, jnp.int32(0))

    pltpu.sync_copy(o_v, out_hbm.at[pl.ds(base, BPW)])


def kernel(user_ids, last_items, pre_items, user_table, item_table,
           global_transition, item_biases):
    uid = user_ids.astype(jnp.int32).reshape(B // 128, 128)
    lid = last_items.astype(jnp.int32).reshape(B // 128, 128)
    pid = pre_items.astype(jnp.int32).reshape(B // 128, 128)
    ut3 = user_table.reshape(125000, 8, D)
    it3 = item_table.reshape(125000, 8, D)
    bias_flat = item_biases.reshape(-1)
    bias2 = jnp.pad(bias_flat, (0, BROWS * 128 - bias_flat.shape[0]))
    bias2 = bias2.reshape(BROWS, 128)
    return _trans_rec_sc(uid, lid, pid, ut3, it3, global_transition, bias2)
